# baseline (speedup 1.0000x reference)
import functools

import jax
import jax.numpy as jnp
from jax import lax
from jax.experimental import pallas as pl
from jax.experimental.pallas import tpu as pltpu
from jax.experimental.pallas import tpu_sc as plsc

_EPS = 1e-5
_NBLK = 10
_NC, _NS = 2, 16
_NT = _NC * _NS
_N = 50000
_E = 800000
_NB = 1568
_NPAD = _NT * _NB
_CH = 128
_CAP = 16 + _E
_W = 8000
_CB = 4096 + 64
_NEGINF = float("-inf")

_sc_params = pltpu.CompilerParams(needs_layout_passes=False,
                                  use_tc_tiling_on_sc=False)


def _wid():
    return lax.axis_index("s") * _NC + lax.axis_index("c")


def _bucketize_body(src_hbm, dst_hbm, lsrc_hbm, ldl_hbm, sbuf, dbuf, csrc, cdl):
    wid = _wid()
    lo = wid * _NB
    rowbase = wid * _CAP
    iota = lax.iota(jnp.int32, 16)
    nwin = _E // _W

    def window(w, carry):
        pltpu.sync_copy(src_hbm.at[pl.ds(w * _W, _W)], sbuf)
        pltpu.sync_copy(dst_hbm.at[pl.ds(w * _W, _W)], dbuf)

        def step(i, c):
            cur, gofs = c
            for u in range(2):
                d = dbuf[pl.ds((2 * i + u) * 16, 16)]
                s = sbuf[pl.ds((2 * i + u) * 16, 16)]
                m = (d >= lo) & (d < lo + _NB)
                c0 = plsc.all_reduce_population_count(m)[0]

                @pl.when(c0 > 0)
                def _(cur=cur, d=d, s=s, m=m):
                    cp = plsc.cumsum(m.astype(jnp.int32)) + (cur - 1)
                    plsc.store_scatter(csrc, [cp], s, mask=m)
                    plsc.store_scatter(cdl, [cp], d - lo, mask=m)

                cur = cur + c0
            flush = cur >= 4096

            @pl.when(flush)
            def _():
                pltpu.sync_copy(csrc.at[pl.ds(0, 4096)],
                                lsrc_hbm.at[pl.ds(pl.multiple_of(rowbase + gofs, 16), 4096)])
                pltpu.sync_copy(cdl.at[pl.ds(0, 4096)],
                                ldl_hbm.at[pl.ds(pl.multiple_of(rowbase + gofs, 16), 4096)])
                csrc[pl.ds(0, 16)] = csrc[pl.ds(4096, 16)]
                cdl[pl.ds(0, 16)] = cdl[pl.ds(4096, 16)]
                csrc[pl.ds(16, 16)] = csrc[pl.ds(4112, 16)]
                cdl[pl.ds(16, 16)] = cdl[pl.ds(4112, 16)]

            cur = jnp.where(flush, cur - 4096, cur)
            gofs = jnp.where(flush, gofs + 4096, gofs)
            return cur, gofs

        return lax.fori_loop(0, _W // 32, step, carry)

    cur, gofs = lax.fori_loop(0, nwin, window, (jnp.int32(0), jnp.int32(16)))

    total = gofs - 16 + cur
    padded = ((total + _CH - 1) // _CH) * _CH
    pc = padded - (gofs - 16)
    b0 = (cur // 16) * 16
    msk = iota >= (cur - b0)
    plsc.store_scatter(csrc, [b0 + iota], jnp.zeros((16,), jnp.int32), mask=msk)
    plsc.store_scatter(cdl, [b0 + iota], jnp.full((16,), _NB, jnp.int32),
                       mask=msk)

    def fill(j, _):
        ofs = b0 + 16 + j * 16
        csrc[pl.ds(ofs, 16)] = jnp.zeros((16,), jnp.int32)
        cdl[pl.ds(ofs, 16)] = jnp.full((16,), _NB, jnp.int32)
        return 0

    lax.fori_loop(0, jnp.maximum(0, (pc - b0 - 16) // 16), fill, 0)

    def flush16(j, _):
        pltpu.sync_copy(csrc.at[pl.ds(j * 16, 16)],
                        lsrc_hbm.at[pl.ds(pl.multiple_of(rowbase + gofs + j * 16, 16), 16)])
        pltpu.sync_copy(cdl.at[pl.ds(j * 16, 16)],
                        ldl_hbm.at[pl.ds(pl.multiple_of(rowbase + gofs + j * 16, 16), 16)])
        return 0

    lax.fori_loop(0, pc // 16, flush16, 0)

    csrc[pl.ds(0, 16)] = jnp.full((16,), padded, jnp.int32)
    pltpu.sync_copy(csrc.at[pl.ds(0, 16)], ldl_hbm.at[pl.ds(pl.multiple_of(rowbase, 16), 16)])


def _bucketize(src, dst):
    mesh = plsc.VectorSubcoreMesh(core_axis_name="c", subcore_axis_name="s")
    f = pl.kernel(
        _bucketize_body,
        out_type=[jax.ShapeDtypeStruct((_NT * _CAP,), jnp.int32),
                  jax.ShapeDtypeStruct((_NT * _CAP,), jnp.int32)],
        mesh=mesh,
        scratch_types=[
            pltpu.VMEM((_W,), jnp.int32),
            pltpu.VMEM((_W,), jnp.int32),
            pltpu.VMEM((_CB,), jnp.int32),
            pltpu.VMEM((_CB,), jnp.int32),
        ],
        compiler_params=_sc_params,
        name="sc_bucketize",
    )
    return f(src, dst)


def _segmax_body(cf, feat_hbm, lsrc_hbm, ldl_hbm, out_hbm, sidx0, sidx1,
                 dlb0, dlb1, rows0, rows1, acc, hdr, sem0, sem1):
    wid = _wid()
    lo = wid * _NB
    rowbase = wid * _CAP
    nacc = (_NB + 1) * cf

    def init(j, _):
        acc[pl.ds(j * 16, 16)] = jnp.full((16,), _NEGINF, jnp.float32)
        return 0

    lax.fori_loop(0, (nacc + 15) // 16, init, 0)

    pltpu.sync_copy(ldl_hbm.at[pl.ds(pl.multiple_of(rowbase, 16), 16)], hdr)
    M = hdr[...][0]
    nch = M // _CH

    def fetch(c, sidx, dlb, rows, sem):
        base = pl.multiple_of(rowbase + 16 + c * _CH, 16)
        pltpu.sync_copy(lsrc_hbm.at[pl.ds(base, _CH)], sidx)
        pltpu.sync_copy(ldl_hbm.at[pl.ds(base, _CH)], dlb)
        pltpu.async_copy(feat_hbm.at[sidx], rows, sem)

    def compute(dlb, rows):
        def grp(g, _):
            dlv = dlb[pl.ds(g * 16, 16)] * cf
            for e in range(16):
                aof = dlv[e]
                ro = g * 16 + e
                for k in range(cf // 16):
                    a = acc[pl.ds(aof + k * 16, 16)]
                    v = rows[ro, pl.ds(k * 16, 16)]
                    acc[pl.ds(aof + k * 16, 16)] = jnp.maximum(a, v)
            return 0

        lax.fori_loop(0, _CH // 16, grp, 0)

    @pl.when(nch > 0)
    def _():
        fetch(0, sidx0, dlb0, rows0, sem0)

    def pair(p, _):
        e = 2 * p
        pltpu.make_async_copy(feat_hbm.at[sidx0], rows0, sem0).wait()

        @pl.when(e + 1 < nch)
        def _():
            fetch(e + 1, sidx1, dlb1, rows1, sem1)
        compute(dlb0, rows0)

        @pl.when(e + 1 < nch)
        def _():
            pltpu.make_async_copy(feat_hbm.at[sidx1], rows1, sem1).wait()

            @pl.when(e + 2 < nch)
            def _():
                fetch(e + 2, sidx0, dlb0, rows0, sem0)
            compute(dlb1, rows1)
        return 0

    lax.fori_loop(0, (nch + 1) // 2, pair, 0)
    pltpu.sync_copy(acc.at[pl.ds(0, _NB * cf)],
                    out_hbm.at[pl.ds(pl.multiple_of(lo * cf, 16), _NB * cf)])


def _segmax(feat, lsrc, ldl):
    cf = feat.shape[1]
    mesh = plsc.VectorSubcoreMesh(core_axis_name="c", subcore_axis_name="s")
    f = pl.kernel(
        functools.partial(_segmax_body, cf),
        out_type=jax.ShapeDtypeStruct((_NPAD * cf,), jnp.float32),
        mesh=mesh,
        scratch_types=[
            pltpu.VMEM((_CH,), jnp.int32),
            pltpu.VMEM((_CH,), jnp.int32),
            pltpu.VMEM((_CH,), jnp.int32),
            pltpu.VMEM((_CH,), jnp.int32),
            pltpu.VMEM((_CH, cf), jnp.float32),
            pltpu.VMEM((_CH, cf), jnp.float32),
            pltpu.VMEM(((_NB + 1) * cf,), jnp.float32),
            pltpu.VMEM((16,), jnp.int32),
            pltpu.SemaphoreType.DMA,
            pltpu.SemaphoreType.DMA,
        ],
        compiler_params=_sc_params,
        name=f"sc_segmax{cf}",
    )
    return f(feat, lsrc, ldl).reshape(_NPAD, cf)[:_N]


def _mm_body(nblk, do_stats, aggF_ref, aggP_ref, pos_ref, W1_ref, W2_ref,
             b_ref, o_ref, stats_ref, ssum, ssq):
    i = pl.program_id(0)
    F = aggF_ref[...]
    F = jnp.where(jnp.isfinite(F), F, 0.0)
    P = aggP_ref[...]
    P = jnp.where(jnp.isfinite(P), P - pos_ref[...], 0.0)
    raw = (jnp.dot(F, W1_ref[...], preferred_element_type=jnp.float32)
           + jnp.dot(P, W2_ref[...], preferred_element_type=jnp.float32)
           + b_ref[...])
    o_ref[...] = raw
    if do_stats:
        @pl.when(i == 0)
        def _():
            ssum[...] = jnp.zeros_like(ssum)
            ssq[...] = jnp.zeros_like(ssq)
        ssum[...] += jnp.sum(raw, axis=0, keepdims=True)
        ssq[...] += jnp.sum(raw * raw, axis=0, keepdims=True)

        @pl.when(i == nblk - 1)
        def _():
            stats_ref[0:1, :] = ssum[...]
            stats_ref[1:2, :] = ssq[...]


def _bn_body(n_rows, raw_ref, stats_ref, g_ref, beta_ref, o_ref):
    mu = stats_ref[0:1, :] / n_rows
    var = stats_ref[1:2, :] / n_rows - mu * mu
    rstd = jax.lax.rsqrt(var + _EPS)
    out = (raw_ref[...] - mu) * rstd * g_ref[...] + beta_ref[...]
    o_ref[...] = jnp.maximum(out, 0.0)


def _conv(aggF, aggP, pos2d, W, b, g, beta, do_bn, cout_pad):
    N = aggF.shape[0]
    nblk = _NBLK
    rb = N // nblk
    cout = W.shape[0]
    W1 = jnp.zeros((64, cout_pad), jnp.float32).at[:, :cout].set(W[:, :64].T)
    W2 = jnp.zeros((2, cout_pad), jnp.float32).at[:, :cout].set(W[:, 64:66].T)
    bp = jnp.zeros((1, cout_pad), jnp.float32).at[:, :cout].set(b[None, :])

    row_spec = lambda c: pl.BlockSpec((rb, c), lambda i: (i, 0))
    rep_spec = lambda r, c: pl.BlockSpec((r, c), lambda i: (0, 0))
    raw, stats = pl.pallas_call(
        functools.partial(_mm_body, nblk, do_bn),
        grid=(nblk,),
        in_specs=[row_spec(64), row_spec(2), row_spec(2),
                  rep_spec(64, cout_pad), rep_spec(2, cout_pad),
                  rep_spec(1, cout_pad)],
        out_specs=[row_spec(cout_pad), rep_spec(2, cout_pad)],
        out_shape=[jax.ShapeDtypeStruct((N, cout_pad), jnp.float32),
                   jax.ShapeDtypeStruct((2, cout_pad), jnp.float32)],
        scratch_shapes=[pltpu.VMEM((1, cout_pad), jnp.float32),
                        pltpu.VMEM((1, cout_pad), jnp.float32)],
    )(aggF, aggP, pos2d, W1, W2, bp)
    if not do_bn:
        return raw
    gp = jnp.ones((1, cout_pad), jnp.float32).at[:, :cout].set(g[None, :])
    betap = jnp.zeros((1, cout_pad), jnp.float32).at[:, :cout].set(beta[None, :])
    out = pl.pallas_call(
        functools.partial(_bn_body, float(N)),
        grid=(nblk,),
        in_specs=[row_spec(cout_pad), rep_spec(2, cout_pad),
                  rep_spec(1, cout_pad), rep_spec(1, cout_pad)],
        out_specs=row_spec(cout_pad),
        out_shape=jax.ShapeDtypeStruct((N, cout_pad), jnp.float32),
    )(raw, stats, gp, betap)
    return out


def kernel(x, pos, edge_index, W_stem, b_stem, g_stem, beta_stem, W_c1, b_c1,
           g_c1, beta_c1, W_c2, b_c2, g_c2, beta_c2, W_regr, b_regr, W_cls,
           b_cls, W_obj, b_obj):
    pos2d = pos[:, :2]
    src = edge_index[0]
    dst = edge_index[1]

    lsrc, ldl = _bucketize(src, dst)
    pos16 = jnp.zeros((_N, 16), jnp.float32).at[:, :2].set(pos2d)
    aggP = _segmax(pos16, lsrc, ldl)[:, :2]
    aggX = _segmax(x, lsrc, ldl)
    h = _conv(aggX, aggP, pos2d, W_stem, b_stem, g_stem, beta_stem, True, 64)
    aggH = _segmax(h, lsrc, ldl)
    x1 = _conv(aggH, aggP, pos2d, W_c1, b_c1, g_c1, beta_c1, True, 64)
    x2 = _conv(aggH, aggP, pos2d, W_c2, b_c2, g_c2, beta_c2, True, 64)
    aggX1 = _segmax(x1, lsrc, ldl)
    aggX2 = _segmax(x2, lsrc, ldl)
    W_ro = jnp.concatenate([W_regr, W_obj], axis=0)
    b_ro = jnp.concatenate([b_regr, b_obj], axis=0)
    regobj = _conv(aggX1, aggP, pos2d, W_ro, b_ro, None, None, False, 128)
    cls = _conv(aggX2, aggP, pos2d, W_cls, b_cls, None, None, False, 128)
    return cls[:, :101], regobj[:, :4], regobj[:, 4:5]


# speedup vs baseline: 1.1927x; 1.1927x over previous
import functools

import jax
import jax.numpy as jnp
from jax import lax
from jax.experimental import pallas as pl
from jax.experimental.pallas import tpu as pltpu
from jax.experimental.pallas import tpu_sc as plsc

_EPS = 1e-5
_NBLK = 10
_NC, _NS = 2, 16
_NT = _NC * _NS
_N = 50000
_E = 800000
_NB = 1568
_NPAD = _NT * _NB
_CH = 128
_CAP = 16 + _E
_W = 8000
_CB = 4096 + 64
_NEGINF = float("-inf")

_sc_params = pltpu.CompilerParams(needs_layout_passes=False,
                                  use_tc_tiling_on_sc=False)


def _wid():
    return lax.axis_index("s") * _NC + lax.axis_index("c")


def _bucketize_body(src_hbm, dst_hbm, lsrc_hbm, ldl_hbm, sbuf, dbuf, csrc, cdl):
    wid = _wid()
    lo = wid * _NB
    rowbase = wid * _CAP
    iota = lax.iota(jnp.int32, 16)
    nwin = _E // _W

    def window(w, carry):
        pltpu.sync_copy(src_hbm.at[pl.ds(w * _W, _W)], sbuf)
        pltpu.sync_copy(dst_hbm.at[pl.ds(w * _W, _W)], dbuf)

        def step(i, c):
            cur, gofs = c
            for u in range(2):
                d = dbuf[pl.ds((2 * i + u) * 16, 16)]
                s = sbuf[pl.ds((2 * i + u) * 16, 16)]
                m = (d >= lo) & (d < lo + _NB)
                c0 = plsc.all_reduce_population_count(m)[0]

                @pl.when(c0 > 0)
                def _(cur=cur, d=d, s=s, m=m):
                    cp = plsc.cumsum(m.astype(jnp.int32)) + (cur - 1)
                    plsc.store_scatter(csrc, [cp], s, mask=m)
                    plsc.store_scatter(cdl, [cp], d - lo, mask=m)

                cur = cur + c0
            flush = cur >= 4096

            @pl.when(flush)
            def _():
                pltpu.sync_copy(csrc.at[pl.ds(0, 4096)],
                                lsrc_hbm.at[pl.ds(pl.multiple_of(rowbase + gofs, 16), 4096)])
                pltpu.sync_copy(cdl.at[pl.ds(0, 4096)],
                                ldl_hbm.at[pl.ds(pl.multiple_of(rowbase + gofs, 16), 4096)])
                csrc[pl.ds(0, 16)] = csrc[pl.ds(4096, 16)]
                cdl[pl.ds(0, 16)] = cdl[pl.ds(4096, 16)]
                csrc[pl.ds(16, 16)] = csrc[pl.ds(4112, 16)]
                cdl[pl.ds(16, 16)] = cdl[pl.ds(4112, 16)]

            cur = jnp.where(flush, cur - 4096, cur)
            gofs = jnp.where(flush, gofs + 4096, gofs)
            return cur, gofs

        return lax.fori_loop(0, _W // 32, step, carry)

    cur, gofs = lax.fori_loop(0, nwin, window, (jnp.int32(0), jnp.int32(16)))

    total = gofs - 16 + cur
    padded = ((total + _CH - 1) // _CH) * _CH
    pc = padded - (gofs - 16)
    b0 = (cur // 16) * 16
    msk = iota >= (cur - b0)
    plsc.store_scatter(csrc, [b0 + iota], jnp.zeros((16,), jnp.int32), mask=msk)
    plsc.store_scatter(cdl, [b0 + iota], jnp.full((16,), _NB, jnp.int32),
                       mask=msk)

    def fill(j, _):
        ofs = b0 + 16 + j * 16
        csrc[pl.ds(ofs, 16)] = jnp.zeros((16,), jnp.int32)
        cdl[pl.ds(ofs, 16)] = jnp.full((16,), _NB, jnp.int32)
        return 0

    lax.fori_loop(0, jnp.maximum(0, (pc - b0 - 16) // 16), fill, 0)

    def flush16(j, _):
        pltpu.sync_copy(csrc.at[pl.ds(j * 16, 16)],
                        lsrc_hbm.at[pl.ds(pl.multiple_of(rowbase + gofs + j * 16, 16), 16)])
        pltpu.sync_copy(cdl.at[pl.ds(j * 16, 16)],
                        ldl_hbm.at[pl.ds(pl.multiple_of(rowbase + gofs + j * 16, 16), 16)])
        return 0

    lax.fori_loop(0, pc // 16, flush16, 0)

    csrc[pl.ds(0, 16)] = jnp.full((16,), padded, jnp.int32)
    pltpu.sync_copy(csrc.at[pl.ds(0, 16)], ldl_hbm.at[pl.ds(pl.multiple_of(rowbase, 16), 16)])


def _bucketize(src, dst):
    mesh = plsc.VectorSubcoreMesh(core_axis_name="c", subcore_axis_name="s")
    f = pl.kernel(
        _bucketize_body,
        out_type=[jax.ShapeDtypeStruct((_NT * _CAP,), jnp.int32),
                  jax.ShapeDtypeStruct((_NT * _CAP,), jnp.int32)],
        mesh=mesh,
        scratch_types=[
            pltpu.VMEM((_W,), jnp.int32),
            pltpu.VMEM((_W,), jnp.int32),
            pltpu.VMEM((_CB,), jnp.int32),
            pltpu.VMEM((_CB,), jnp.int32),
        ],
        compiler_params=_sc_params,
        name="sc_bucketize",
    )
    return f(src, dst)


def _segmax_body(cf, feat_hbm, lsrc_hbm, ldl_hbm, out_hbm, sidx0, sidx1,
                 dlb0, dlb1, rows0, rows1, acc, hdr, sem0, sem1):
    wid = _wid()
    lo = wid * _NB
    rowbase = wid * _CAP
    nacc = (_NB + 1) * cf

    def init(j, _):
        acc[pl.ds(j * 16, 16)] = jnp.full((16,), _NEGINF, jnp.float32)
        return 0

    lax.fori_loop(0, (nacc + 15) // 16, init, 0)

    pltpu.sync_copy(ldl_hbm.at[pl.ds(pl.multiple_of(rowbase, 16), 16)], hdr)
    M = hdr[...][0]
    nch = M // _CH

    def fetch(c, sidx, dlb, rows, sem):
        base = pl.multiple_of(rowbase + 16 + c * _CH, 16)
        pltpu.sync_copy(lsrc_hbm.at[pl.ds(base, _CH)], sidx)
        pltpu.sync_copy(ldl_hbm.at[pl.ds(base, _CH)], dlb)
        pltpu.async_copy(feat_hbm.at[sidx], rows, sem)

    def compute(dlb, rows):
        def grp(g, _):
            dlv = dlb[pl.ds(g * 16, 16)] * cf
            for e in range(16):
                aof = dlv[e]
                ro = g * 16 + e
                nk = cf // 16
                avs = [acc[pl.ds(aof + k * 16, 16)] for k in range(nk)]
                vvs = [rows[ro, pl.ds(k * 16, 16)] for k in range(nk)]
                for k in range(nk):
                    acc[pl.ds(aof + k * 16, 16)] = jnp.maximum(avs[k], vvs[k])
            return 0

        lax.fori_loop(0, _CH // 16, grp, 0)

    @pl.when(nch > 0)
    def _():
        fetch(0, sidx0, dlb0, rows0, sem0)

    def pair(p, _):
        e = 2 * p
        pltpu.make_async_copy(feat_hbm.at[sidx0], rows0, sem0).wait()

        @pl.when(e + 1 < nch)
        def _():
            fetch(e + 1, sidx1, dlb1, rows1, sem1)
        compute(dlb0, rows0)

        @pl.when(e + 1 < nch)
        def _():
            pltpu.make_async_copy(feat_hbm.at[sidx1], rows1, sem1).wait()

            @pl.when(e + 2 < nch)
            def _():
                fetch(e + 2, sidx0, dlb0, rows0, sem0)
            compute(dlb1, rows1)
        return 0

    lax.fori_loop(0, (nch + 1) // 2, pair, 0)
    pltpu.sync_copy(acc.at[pl.ds(0, _NB * cf)],
                    out_hbm.at[pl.ds(pl.multiple_of(lo * cf, 16), _NB * cf)])


def _segmax(feat, lsrc, ldl):
    cf = feat.shape[1]
    mesh = plsc.VectorSubcoreMesh(core_axis_name="c", subcore_axis_name="s")
    f = pl.kernel(
        functools.partial(_segmax_body, cf),
        out_type=jax.ShapeDtypeStruct((_NPAD * cf,), jnp.float32),
        mesh=mesh,
        scratch_types=[
            pltpu.VMEM((_CH,), jnp.int32),
            pltpu.VMEM((_CH,), jnp.int32),
            pltpu.VMEM((_CH,), jnp.int32),
            pltpu.VMEM((_CH,), jnp.int32),
            pltpu.VMEM((_CH, cf), jnp.float32),
            pltpu.VMEM((_CH, cf), jnp.float32),
            pltpu.VMEM(((_NB + 1) * cf,), jnp.float32),
            pltpu.VMEM((16,), jnp.int32),
            pltpu.SemaphoreType.DMA,
            pltpu.SemaphoreType.DMA,
        ],
        compiler_params=_sc_params,
        name=f"sc_segmax{cf}",
    )
    return f(feat, lsrc, ldl).reshape(_NPAD, cf)[:_N]


def _mm_body(nblk, do_stats, aggF_ref, aggP_ref, pos_ref, W1_ref, W2_ref,
             b_ref, o_ref, stats_ref, ssum, ssq):
    i = pl.program_id(0)
    F = aggF_ref[...]
    F = jnp.where(jnp.isfinite(F), F, 0.0)
    P = aggP_ref[...]
    P = jnp.where(jnp.isfinite(P), P - pos_ref[...], 0.0)
    raw = (jnp.dot(F, W1_ref[...], preferred_element_type=jnp.float32)
           + jnp.dot(P, W2_ref[...], preferred_element_type=jnp.float32)
           + b_ref[...])
    o_ref[...] = raw
    if do_stats:
        @pl.when(i == 0)
        def _():
            ssum[...] = jnp.zeros_like(ssum)
            ssq[...] = jnp.zeros_like(ssq)
        ssum[...] += jnp.sum(raw, axis=0, keepdims=True)
        ssq[...] += jnp.sum(raw * raw, axis=0, keepdims=True)

        @pl.when(i == nblk - 1)
        def _():
            stats_ref[0:1, :] = ssum[...]
            stats_ref[1:2, :] = ssq[...]


def _bn_body(n_rows, raw_ref, stats_ref, g_ref, beta_ref, o_ref):
    mu = stats_ref[0:1, :] / n_rows
    var = stats_ref[1:2, :] / n_rows - mu * mu
    rstd = jax.lax.rsqrt(var + _EPS)
    out = (raw_ref[...] - mu) * rstd * g_ref[...] + beta_ref[...]
    o_ref[...] = jnp.maximum(out, 0.0)


def _conv(aggF, aggP, pos2d, W, b, g, beta, do_bn, cout_pad):
    N = aggF.shape[0]
    nblk = _NBLK
    rb = N // nblk
    cout = W.shape[0]
    W1 = jnp.zeros((64, cout_pad), jnp.float32).at[:, :cout].set(W[:, :64].T)
    W2 = jnp.zeros((2, cout_pad), jnp.float32).at[:, :cout].set(W[:, 64:66].T)
    bp = jnp.zeros((1, cout_pad), jnp.float32).at[:, :cout].set(b[None, :])

    row_spec = lambda c: pl.BlockSpec((rb, c), lambda i: (i, 0))
    rep_spec = lambda r, c: pl.BlockSpec((r, c), lambda i: (0, 0))
    raw, stats = pl.pallas_call(
        functools.partial(_mm_body, nblk, do_bn),
        grid=(nblk,),
        in_specs=[row_spec(64), row_spec(2), row_spec(2),
                  rep_spec(64, cout_pad), rep_spec(2, cout_pad),
                  rep_spec(1, cout_pad)],
        out_specs=[row_spec(cout_pad), rep_spec(2, cout_pad)],
        out_shape=[jax.ShapeDtypeStruct((N, cout_pad), jnp.float32),
                   jax.ShapeDtypeStruct((2, cout_pad), jnp.float32)],
        scratch_shapes=[pltpu.VMEM((1, cout_pad), jnp.float32),
                        pltpu.VMEM((1, cout_pad), jnp.float32)],
    )(aggF, aggP, pos2d, W1, W2, bp)
    if not do_bn:
        return raw
    gp = jnp.ones((1, cout_pad), jnp.float32).at[:, :cout].set(g[None, :])
    betap = jnp.zeros((1, cout_pad), jnp.float32).at[:, :cout].set(beta[None, :])
    out = pl.pallas_call(
        functools.partial(_bn_body, float(N)),
        grid=(nblk,),
        in_specs=[row_spec(cout_pad), rep_spec(2, cout_pad),
                  rep_spec(1, cout_pad), rep_spec(1, cout_pad)],
        out_specs=row_spec(cout_pad),
        out_shape=jax.ShapeDtypeStruct((N, cout_pad), jnp.float32),
    )(raw, stats, gp, betap)
    return out


def kernel(x, pos, edge_index, W_stem, b_stem, g_stem, beta_stem, W_c1, b_c1,
           g_c1, beta_c1, W_c2, b_c2, g_c2, beta_c2, W_regr, b_regr, W_cls,
           b_cls, W_obj, b_obj):
    pos2d = pos[:, :2]
    src = edge_index[0]
    dst = edge_index[1]

    lsrc, ldl = _bucketize(src, dst)
    pos16 = jnp.zeros((_N, 16), jnp.float32).at[:, :2].set(pos2d)
    aggP = _segmax(pos16, lsrc, ldl)[:, :2]
    aggX = _segmax(x, lsrc, ldl)
    h = _conv(aggX, aggP, pos2d, W_stem, b_stem, g_stem, beta_stem, True, 64)
    aggH = _segmax(h, lsrc, ldl)
    x1 = _conv(aggH, aggP, pos2d, W_c1, b_c1, g_c1, beta_c1, True, 64)
    x2 = _conv(aggH, aggP, pos2d, W_c2, b_c2, g_c2, beta_c2, True, 64)
    aggX1 = _segmax(x1, lsrc, ldl)
    aggX2 = _segmax(x2, lsrc, ldl)
    W_ro = jnp.concatenate([W_regr, W_obj], axis=0)
    b_ro = jnp.concatenate([b_regr, b_obj], axis=0)
    regobj = _conv(aggX1, aggP, pos2d, W_ro, b_ro, None, None, False, 128)
    cls = _conv(aggX2, aggP, pos2d, W_cls, b_cls, None, None, False, 128)
    return cls[:, :101], regobj[:, :4], regobj[:, 4:5]


# speedup vs baseline: 1.1964x; 1.0030x over previous
import functools

import jax
import jax.numpy as jnp
from jax import lax
from jax.experimental import pallas as pl
from jax.experimental.pallas import tpu as pltpu
from jax.experimental.pallas import tpu_sc as plsc

_EPS = 1e-5
_NBLK = 10
_NC, _NS = 2, 16
_NT = _NC * _NS
_N = 50000
_E = 800000
_NB = 1568
_NPAD = _NT * _NB
_CH = 128
_CAP = 16 + _E
_W = 8000
_CB = 4096 + 64
_NEGINF = float("-inf")

_sc_params = pltpu.CompilerParams(needs_layout_passes=False,
                                  use_tc_tiling_on_sc=False)


def _wid():
    return lax.axis_index("s") * _NC + lax.axis_index("c")


def _bucketize_body(src_hbm, dst_hbm, lsrc_hbm, ldl_hbm, sbuf, dbuf, csrc, cdl):
    wid = _wid()
    lo = wid * _NB
    rowbase = wid * _CAP
    iota = lax.iota(jnp.int32, 16)
    nwin = _E // _W

    def window(w, carry):
        ww = lax.rem(w + wid * 3, nwin)
        wof = pl.multiple_of(ww * _W, 16)
        pltpu.sync_copy(src_hbm.at[pl.ds(wof, _W)], sbuf)
        pltpu.sync_copy(dst_hbm.at[pl.ds(wof, _W)], dbuf)

        def step(i, c):
            cur, gofs = c
            for u in range(2):
                d = dbuf[pl.ds((2 * i + u) * 16, 16)]
                s = sbuf[pl.ds((2 * i + u) * 16, 16)]
                m = (d >= lo) & (d < lo + _NB)
                c0 = plsc.all_reduce_population_count(m)[0]

                @pl.when(c0 > 0)
                def _(cur=cur, d=d, s=s, m=m):
                    cp = plsc.cumsum(m.astype(jnp.int32)) + (cur - 1)
                    plsc.store_scatter(csrc, [cp], s, mask=m)
                    plsc.store_scatter(cdl, [cp], d - lo, mask=m)

                cur = cur + c0
            flush = cur >= 4096

            @pl.when(flush)
            def _():
                pltpu.sync_copy(csrc.at[pl.ds(0, 4096)],
                                lsrc_hbm.at[pl.ds(pl.multiple_of(rowbase + gofs, 16), 4096)])
                pltpu.sync_copy(cdl.at[pl.ds(0, 4096)],
                                ldl_hbm.at[pl.ds(pl.multiple_of(rowbase + gofs, 16), 4096)])
                csrc[pl.ds(0, 16)] = csrc[pl.ds(4096, 16)]
                cdl[pl.ds(0, 16)] = cdl[pl.ds(4096, 16)]
                csrc[pl.ds(16, 16)] = csrc[pl.ds(4112, 16)]
                cdl[pl.ds(16, 16)] = cdl[pl.ds(4112, 16)]

            cur = jnp.where(flush, cur - 4096, cur)
            gofs = jnp.where(flush, gofs + 4096, gofs)
            return cur, gofs

        return lax.fori_loop(0, _W // 32, step, carry)

    cur, gofs = lax.fori_loop(0, nwin, window, (jnp.int32(0), jnp.int32(16)))

    total = gofs - 16 + cur
    padded = ((total + _CH - 1) // _CH) * _CH
    pc = padded - (gofs - 16)
    b0 = (cur // 16) * 16
    msk = iota >= (cur - b0)
    plsc.store_scatter(csrc, [b0 + iota], jnp.zeros((16,), jnp.int32), mask=msk)
    plsc.store_scatter(cdl, [b0 + iota], jnp.full((16,), _NB, jnp.int32),
                       mask=msk)

    def fill(j, _):
        ofs = b0 + 16 + j * 16
        csrc[pl.ds(ofs, 16)] = jnp.zeros((16,), jnp.int32)
        cdl[pl.ds(ofs, 16)] = jnp.full((16,), _NB, jnp.int32)
        return 0

    lax.fori_loop(0, jnp.maximum(0, (pc - b0 - 16) // 16), fill, 0)

    def flush16(j, _):
        pltpu.sync_copy(csrc.at[pl.ds(j * 16, 16)],
                        lsrc_hbm.at[pl.ds(pl.multiple_of(rowbase + gofs + j * 16, 16), 16)])
        pltpu.sync_copy(cdl.at[pl.ds(j * 16, 16)],
                        ldl_hbm.at[pl.ds(pl.multiple_of(rowbase + gofs + j * 16, 16), 16)])
        return 0

    lax.fori_loop(0, pc // 16, flush16, 0)

    csrc[pl.ds(0, 16)] = jnp.full((16,), padded, jnp.int32)
    pltpu.sync_copy(csrc.at[pl.ds(0, 16)], ldl_hbm.at[pl.ds(pl.multiple_of(rowbase, 16), 16)])


def _bucketize(src, dst):
    mesh = plsc.VectorSubcoreMesh(core_axis_name="c", subcore_axis_name="s")
    f = pl.kernel(
        _bucketize_body,
        out_type=[jax.ShapeDtypeStruct((_NT * _CAP,), jnp.int32),
                  jax.ShapeDtypeStruct((_NT * _CAP,), jnp.int32)],
        mesh=mesh,
        scratch_types=[
            pltpu.VMEM((_W,), jnp.int32),
            pltpu.VMEM((_W,), jnp.int32),
            pltpu.VMEM((_CB,), jnp.int32),
            pltpu.VMEM((_CB,), jnp.int32),
        ],
        compiler_params=_sc_params,
        name="sc_bucketize",
    )
    return f(src, dst)


def _segmax_body(cf, feat_hbm, lsrc_hbm, ldl_hbm, out_hbm, sidx0, sidx1,
                 dlb0, dlb1, rows0, rows1, acc, hdr, sem0, sem1):
    wid = _wid()
    lo = wid * _NB
    rowbase = wid * _CAP
    nacc = (_NB + 1) * cf

    def init(j, _):
        acc[pl.ds(j * 16, 16)] = jnp.full((16,), _NEGINF, jnp.float32)
        return 0

    lax.fori_loop(0, (nacc + 15) // 16, init, 0)

    pltpu.sync_copy(ldl_hbm.at[pl.ds(pl.multiple_of(rowbase, 16), 16)], hdr)
    M = hdr[...][0]
    nch = M // _CH

    def fetch(c, sidx, dlb, rows, sem):
        base = pl.multiple_of(rowbase + 16 + c * _CH, 16)
        pltpu.sync_copy(lsrc_hbm.at[pl.ds(base, _CH)], sidx)
        pltpu.sync_copy(ldl_hbm.at[pl.ds(base, _CH)], dlb)
        pltpu.async_copy(feat_hbm.at[sidx], rows, sem)

    def compute(dlb, rows):
        def grp(g, _):
            dlv = dlb[pl.ds(g * 16, 16)] * cf
            for e in range(16):
                aof = dlv[e]
                ro = g * 16 + e
                nk = cf // 16
                avs = [acc[pl.ds(aof + k * 16, 16)] for k in range(nk)]
                vvs = [rows[ro, pl.ds(k * 16, 16)] for k in range(nk)]
                for k in range(nk):
                    acc[pl.ds(aof + k * 16, 16)] = jnp.maximum(avs[k], vvs[k])
            return 0

        lax.fori_loop(0, _CH // 16, grp, 0)

    @pl.when(nch > 0)
    def _():
        fetch(0, sidx0, dlb0, rows0, sem0)

    def pair(p, _):
        e = 2 * p
        pltpu.make_async_copy(feat_hbm.at[sidx0], rows0, sem0).wait()

        @pl.when(e + 1 < nch)
        def _():
            fetch(e + 1, sidx1, dlb1, rows1, sem1)
        compute(dlb0, rows0)

        @pl.when(e + 1 < nch)
        def _():
            pltpu.make_async_copy(feat_hbm.at[sidx1], rows1, sem1).wait()

            @pl.when(e + 2 < nch)
            def _():
                fetch(e + 2, sidx0, dlb0, rows0, sem0)
            compute(dlb1, rows1)
        return 0

    lax.fori_loop(0, (nch + 1) // 2, pair, 0)
    pltpu.sync_copy(acc.at[pl.ds(0, _NB * cf)],
                    out_hbm.at[pl.ds(pl.multiple_of(lo * cf, 16), _NB * cf)])


def _segmax(feat, lsrc, ldl):
    cf = feat.shape[1]
    mesh = plsc.VectorSubcoreMesh(core_axis_name="c", subcore_axis_name="s")
    f = pl.kernel(
        functools.partial(_segmax_body, cf),
        out_type=jax.ShapeDtypeStruct((_NPAD * cf,), jnp.float32),
        mesh=mesh,
        scratch_types=[
            pltpu.VMEM((_CH,), jnp.int32),
            pltpu.VMEM((_CH,), jnp.int32),
            pltpu.VMEM((_CH,), jnp.int32),
            pltpu.VMEM((_CH,), jnp.int32),
            pltpu.VMEM((_CH, cf), jnp.float32),
            pltpu.VMEM((_CH, cf), jnp.float32),
            pltpu.VMEM(((_NB + 1) * cf,), jnp.float32),
            pltpu.VMEM((16,), jnp.int32),
            pltpu.SemaphoreType.DMA,
            pltpu.SemaphoreType.DMA,
        ],
        compiler_params=_sc_params,
        name=f"sc_segmax{cf}",
    )
    return f(feat, lsrc, ldl).reshape(_NPAD, cf)[:_N]


def _mm_body(nblk, do_stats, aggF_ref, aggP_ref, pos_ref, W1_ref, W2_ref,
             b_ref, o_ref, stats_ref, ssum, ssq):
    i = pl.program_id(0)
    F = aggF_ref[...]
    F = jnp.where(jnp.isfinite(F), F, 0.0)
    P = aggP_ref[...]
    P = jnp.where(jnp.isfinite(P), P - pos_ref[...], 0.0)
    raw = (jnp.dot(F, W1_ref[...], preferred_element_type=jnp.float32)
           + jnp.dot(P, W2_ref[...], preferred_element_type=jnp.float32)
           + b_ref[...])
    o_ref[...] = raw
    if do_stats:
        @pl.when(i == 0)
        def _():
            ssum[...] = jnp.zeros_like(ssum)
            ssq[...] = jnp.zeros_like(ssq)
        ssum[...] += jnp.sum(raw, axis=0, keepdims=True)
        ssq[...] += jnp.sum(raw * raw, axis=0, keepdims=True)

        @pl.when(i == nblk - 1)
        def _():
            stats_ref[0:1, :] = ssum[...]
            stats_ref[1:2, :] = ssq[...]


def _bn_body(n_rows, raw_ref, stats_ref, g_ref, beta_ref, o_ref):
    mu = stats_ref[0:1, :] / n_rows
    var = stats_ref[1:2, :] / n_rows - mu * mu
    rstd = jax.lax.rsqrt(var + _EPS)
    out = (raw_ref[...] - mu) * rstd * g_ref[...] + beta_ref[...]
    o_ref[...] = jnp.maximum(out, 0.0)


def _conv(aggF, aggP, pos2d, W, b, g, beta, do_bn, cout_pad):
    N = aggF.shape[0]
    nblk = _NBLK
    rb = N // nblk
    cout = W.shape[0]
    W1 = jnp.zeros((64, cout_pad), jnp.float32).at[:, :cout].set(W[:, :64].T)
    W2 = jnp.zeros((2, cout_pad), jnp.float32).at[:, :cout].set(W[:, 64:66].T)
    bp = jnp.zeros((1, cout_pad), jnp.float32).at[:, :cout].set(b[None, :])

    row_spec = lambda c: pl.BlockSpec((rb, c), lambda i: (i, 0))
    rep_spec = lambda r, c: pl.BlockSpec((r, c), lambda i: (0, 0))
    raw, stats = pl.pallas_call(
        functools.partial(_mm_body, nblk, do_bn),
        grid=(nblk,),
        in_specs=[row_spec(64), row_spec(2), row_spec(2),
                  rep_spec(64, cout_pad), rep_spec(2, cout_pad),
                  rep_spec(1, cout_pad)],
        out_specs=[row_spec(cout_pad), rep_spec(2, cout_pad)],
        out_shape=[jax.ShapeDtypeStruct((N, cout_pad), jnp.float32),
                   jax.ShapeDtypeStruct((2, cout_pad), jnp.float32)],
        scratch_shapes=[pltpu.VMEM((1, cout_pad), jnp.float32),
                        pltpu.VMEM((1, cout_pad), jnp.float32)],
    )(aggF, aggP, pos2d, W1, W2, bp)
    if not do_bn:
        return raw
    gp = jnp.ones((1, cout_pad), jnp.float32).at[:, :cout].set(g[None, :])
    betap = jnp.zeros((1, cout_pad), jnp.float32).at[:, :cout].set(beta[None, :])
    out = pl.pallas_call(
        functools.partial(_bn_body, float(N)),
        grid=(nblk,),
        in_specs=[row_spec(cout_pad), rep_spec(2, cout_pad),
                  rep_spec(1, cout_pad), rep_spec(1, cout_pad)],
        out_specs=row_spec(cout_pad),
        out_shape=jax.ShapeDtypeStruct((N, cout_pad), jnp.float32),
    )(raw, stats, gp, betap)
    return out


def kernel(x, pos, edge_index, W_stem, b_stem, g_stem, beta_stem, W_c1, b_c1,
           g_c1, beta_c1, W_c2, b_c2, g_c2, beta_c2, W_regr, b_regr, W_cls,
           b_cls, W_obj, b_obj):
    pos2d = pos[:, :2]
    src = edge_index[0]
    dst = edge_index[1]

    lsrc, ldl = _bucketize(src, dst)
    pos16 = jnp.zeros((_N, 16), jnp.float32).at[:, :2].set(pos2d)
    aggP = _segmax(pos16, lsrc, ldl)[:, :2]
    aggX = _segmax(x, lsrc, ldl)
    h = _conv(aggX, aggP, pos2d, W_stem, b_stem, g_stem, beta_stem, True, 64)
    aggH = _segmax(h, lsrc, ldl)
    x1 = _conv(aggH, aggP, pos2d, W_c1, b_c1, g_c1, beta_c1, True, 64)
    x2 = _conv(aggH, aggP, pos2d, W_c2, b_c2, g_c2, beta_c2, True, 64)
    aggX1 = _segmax(x1, lsrc, ldl)
    aggX2 = _segmax(x2, lsrc, ldl)
    W_ro = jnp.concatenate([W_regr, W_obj], axis=0)
    b_ro = jnp.concatenate([b_regr, b_obj], axis=0)
    regobj = _conv(aggX1, aggP, pos2d, W_ro, b_ro, None, None, False, 128)
    cls = _conv(aggX2, aggP, pos2d, W_cls, b_cls, None, None, False, 128)
    return cls[:, :101], regobj[:, :4], regobj[:, 4:5]
